# Initial kernel scaffold; baseline (speedup 1.0000x reference)
import jax
import jax.numpy as jnp
from jax.experimental import pallas as pl


def kernel(x, pos_embedding, start_pos):
    raise NotImplementedError("write your pallas kernel here")


# speedup vs baseline: 1.0467x; 1.0467x over previous
import functools

import jax
import jax.numpy as jnp
from jax.experimental import pallas as pl
from jax.experimental.pallas import tpu as pltpu

D_MODEL = 1024
BS = 256


def _body(sp_ref, x_ref, pe_hbm, o_ref, pe_vmem, sem):
    j = pl.program_id(0)
    b = pl.program_id(1)

    @pl.when(b == 0)
    def _fetch():
        start = pl.multiple_of(sp_ref[0] + j * BS, 8)
        cp = pltpu.make_async_copy(pe_hbm.at[pl.ds(start, BS)], pe_vmem, sem)
        cp.start()
        cp.wait()

    o_ref[0] = x_ref[0] + pe_vmem[...]


@jax.jit
def _pe_add(sp, x, pos_embedding):
    batch, seq, d = x.shape
    grid_spec = pltpu.PrefetchScalarGridSpec(
        num_scalar_prefetch=1,
        grid=(seq // BS, batch),
        in_specs=[
            pl.BlockSpec((1, BS, d), lambda j, b, sp_ref: (b, j, 0)),
            pl.BlockSpec(memory_space=pl.ANY),
        ],
        out_specs=pl.BlockSpec((1, BS, d), lambda j, b, sp_ref: (b, j, 0)),
        scratch_shapes=[
            pltpu.VMEM((BS, d), jnp.float32),
            pltpu.SemaphoreType.DMA,
        ],
    )
    return pl.pallas_call(
        _body,
        grid_spec=grid_spec,
        out_shape=jax.ShapeDtypeStruct(x.shape, x.dtype),
    )(sp, x, pos_embedding)


def kernel(x, pos_embedding, start_pos):
    sp = jnp.atleast_1d(jnp.asarray(start_pos, dtype=jnp.int32))
    return _pe_add(sp, x, pos_embedding)


# speedup vs baseline: 1.3205x; 1.2616x over previous
import jax
import jax.numpy as jnp
from jax.experimental import pallas as pl
from jax.experimental.pallas import tpu as pltpu

D_MODEL = 1024
BS = 256


def _copy(sp_ref, pe_hbm, pe_vmem, sem, j, nj):
    @pl.when(j < nj)
    def _():
        start = pl.multiple_of(sp_ref[0] + j * BS, 8)
        pltpu.make_async_copy(
            pe_hbm.at[pl.ds(start, BS)], pe_vmem.at[j % 2], sem.at[j % 2]
        ).start()


def _body(sp_ref, x_ref, pe_hbm, o_ref, pe_vmem, sem):
    j = pl.program_id(0)
    b = pl.program_id(1)
    nj = pl.num_programs(0)

    @pl.when(b == 0)
    def _fetch():
        @pl.when(j == 0)
        def _prologue():
            _copy(sp_ref, pe_hbm, pe_vmem, sem, 0, nj)

        _copy(sp_ref, pe_hbm, pe_vmem, sem, j + 1, nj)
        pltpu.make_async_copy(
            pe_hbm.at[pl.ds(0, BS)], pe_vmem.at[j % 2], sem.at[j % 2]
        ).wait()

    o_ref[0] = x_ref[0] + pe_vmem[j % 2]


@jax.jit
def _pe_add(sp, x, pos_embedding):
    batch, seq, d = x.shape
    grid_spec = pltpu.PrefetchScalarGridSpec(
        num_scalar_prefetch=1,
        grid=(seq // BS, batch),
        in_specs=[
            pl.BlockSpec((1, BS, d), lambda j, b, sp_ref: (b, j, 0)),
            pl.BlockSpec(memory_space=pl.ANY),
        ],
        out_specs=pl.BlockSpec((1, BS, d), lambda j, b, sp_ref: (b, j, 0)),
        scratch_shapes=[
            pltpu.VMEM((2, BS, d), jnp.float32),
            pltpu.SemaphoreType.DMA((2,)),
        ],
    )
    return pl.pallas_call(
        _body,
        grid_spec=grid_spec,
        out_shape=jax.ShapeDtypeStruct(x.shape, x.dtype),
    )(sp, x, pos_embedding)


def kernel(x, pos_embedding, start_pos):
    sp = jnp.atleast_1d(jnp.asarray(start_pos, dtype=jnp.int32))
    return _pe_add(sp, x, pos_embedding)


# speedup vs baseline: 1.7155x; 1.2992x over previous
import jax
import jax.numpy as jnp
from jax.experimental import pallas as pl
from jax.experimental.pallas import tpu as pltpu

D_MODEL = 1024
BS = 512


def _copy(sp_ref, pe_hbm, pe_vmem, sem, j, nj):
    @pl.when(j < nj)
    def _():
        start = pl.multiple_of(sp_ref[0] + j * BS, 8)
        pltpu.make_async_copy(
            pe_hbm.at[pl.ds(start, BS)], pe_vmem.at[j % 2], sem.at[j % 2]
        ).start()


def _body(sp_ref, x_ref, pe_hbm, o_ref, pe_vmem, sem):
    j = pl.program_id(0)
    b = pl.program_id(1)
    nj = pl.num_programs(0)

    @pl.when(b == 0)
    def _fetch():
        @pl.when(j == 0)
        def _prologue():
            _copy(sp_ref, pe_hbm, pe_vmem, sem, 0, nj)

        _copy(sp_ref, pe_hbm, pe_vmem, sem, j + 1, nj)
        pltpu.make_async_copy(
            pe_hbm.at[pl.ds(0, BS)], pe_vmem.at[j % 2], sem.at[j % 2]
        ).wait()

    o_ref[0] = x_ref[0] + pe_vmem[j % 2]


@jax.jit
def _pe_add(sp, x, pos_embedding):
    batch, seq, d = x.shape
    grid_spec = pltpu.PrefetchScalarGridSpec(
        num_scalar_prefetch=1,
        grid=(seq // BS, batch),
        in_specs=[
            pl.BlockSpec((1, BS, d), lambda j, b, sp_ref: (b, j, 0)),
            pl.BlockSpec(memory_space=pl.ANY),
        ],
        out_specs=pl.BlockSpec((1, BS, d), lambda j, b, sp_ref: (b, j, 0)),
        scratch_shapes=[
            pltpu.VMEM((2, BS, d), jnp.float32),
            pltpu.SemaphoreType.DMA((2,)),
        ],
    )
    return pl.pallas_call(
        _body,
        grid_spec=grid_spec,
        out_shape=jax.ShapeDtypeStruct(x.shape, x.dtype),
    )(sp, x, pos_embedding)


def kernel(x, pos_embedding, start_pos):
    sp = jnp.atleast_1d(jnp.asarray(start_pos, dtype=jnp.int32))
    return _pe_add(sp, x, pos_embedding)


# speedup vs baseline: 1.8628x; 1.0858x over previous
import jax
import jax.numpy as jnp
from jax.experimental import pallas as pl
from jax.experimental.pallas import tpu as pltpu

D_MODEL = 1024
BS = 1024


def _copy(sp_ref, pe_hbm, pe_vmem, sem, j, nj):
    @pl.when(j < nj)
    def _():
        start = pl.multiple_of(sp_ref[0] + j * BS, 8)
        pltpu.make_async_copy(
            pe_hbm.at[pl.ds(start, BS)], pe_vmem.at[j % 2], sem.at[j % 2]
        ).start()


def _body(sp_ref, x_ref, pe_hbm, o_ref, pe_vmem, sem):
    j = pl.program_id(0)
    b = pl.program_id(1)
    nj = pl.num_programs(0)

    @pl.when(b == 0)
    def _fetch():
        @pl.when(j == 0)
        def _prologue():
            _copy(sp_ref, pe_hbm, pe_vmem, sem, 0, nj)

        _copy(sp_ref, pe_hbm, pe_vmem, sem, j + 1, nj)
        pltpu.make_async_copy(
            pe_hbm.at[pl.ds(0, BS)], pe_vmem.at[j % 2], sem.at[j % 2]
        ).wait()

    o_ref[0] = x_ref[0] + pe_vmem[j % 2]


@jax.jit
def _pe_add(sp, x, pos_embedding):
    batch, seq, d = x.shape
    grid_spec = pltpu.PrefetchScalarGridSpec(
        num_scalar_prefetch=1,
        grid=(seq // BS, batch),
        in_specs=[
            pl.BlockSpec((1, BS, d), lambda j, b, sp_ref: (b, j, 0)),
            pl.BlockSpec(memory_space=pl.ANY),
        ],
        out_specs=pl.BlockSpec((1, BS, d), lambda j, b, sp_ref: (b, j, 0)),
        scratch_shapes=[
            pltpu.VMEM((2, BS, d), jnp.float32),
            pltpu.SemaphoreType.DMA((2,)),
        ],
    )
    return pl.pallas_call(
        _body,
        grid_spec=grid_spec,
        out_shape=jax.ShapeDtypeStruct(x.shape, x.dtype),
    )(sp, x, pos_embedding)


def kernel(x, pos_embedding, start_pos):
    sp = jnp.atleast_1d(jnp.asarray(start_pos, dtype=jnp.int32))
    return _pe_add(sp, x, pos_embedding)


# speedup vs baseline: 1.9517x; 1.0477x over previous
import jax
import jax.numpy as jnp
from jax.experimental import pallas as pl
from jax.experimental.pallas import tpu as pltpu

D_MODEL = 1024
BS = 2048


def _copy(sp_ref, pe_hbm, pe_vmem, sem, j, nj):
    @pl.when(j < nj)
    def _():
        start = pl.multiple_of(sp_ref[0] + j * BS, 8)
        pltpu.make_async_copy(
            pe_hbm.at[pl.ds(start, BS)], pe_vmem.at[j % 2], sem.at[j % 2]
        ).start()


def _body(sp_ref, x_ref, pe_hbm, o_ref, pe_vmem, sem):
    j = pl.program_id(0)
    b = pl.program_id(1)
    nj = pl.num_programs(0)

    @pl.when(b == 0)
    def _fetch():
        @pl.when(j == 0)
        def _prologue():
            _copy(sp_ref, pe_hbm, pe_vmem, sem, 0, nj)

        _copy(sp_ref, pe_hbm, pe_vmem, sem, j + 1, nj)
        pltpu.make_async_copy(
            pe_hbm.at[pl.ds(0, BS)], pe_vmem.at[j % 2], sem.at[j % 2]
        ).wait()

    o_ref[0] = x_ref[0] + pe_vmem[j % 2]


@jax.jit
def _pe_add(sp, x, pos_embedding):
    batch, seq, d = x.shape
    grid_spec = pltpu.PrefetchScalarGridSpec(
        num_scalar_prefetch=1,
        grid=(seq // BS, batch),
        in_specs=[
            pl.BlockSpec((1, BS, d), lambda j, b, sp_ref: (b, j, 0)),
            pl.BlockSpec(memory_space=pl.ANY),
        ],
        out_specs=pl.BlockSpec((1, BS, d), lambda j, b, sp_ref: (b, j, 0)),
        scratch_shapes=[
            pltpu.VMEM((2, BS, d), jnp.float32),
            pltpu.SemaphoreType.DMA((2,)),
        ],
    )
    return pl.pallas_call(
        _body,
        grid_spec=grid_spec,
        out_shape=jax.ShapeDtypeStruct(x.shape, x.dtype),
    )(sp, x, pos_embedding)


def kernel(x, pos_embedding, start_pos):
    sp = jnp.atleast_1d(jnp.asarray(start_pos, dtype=jnp.int32))
    return _pe_add(sp, x, pos_embedding)
